# baseline (speedup 1.0000x reference)
import functools

import jax
import jax.numpy as jnp
from jax import lax
from jax.experimental import pallas as pl
from jax.experimental.pallas import tpu as pltpu
from jax.experimental.pallas import tpu_sc as plsc

_NC = 2
_NS = 16
_NW = _NC * _NS
_CH = 4096


@functools.lru_cache(maxsize=None)
def _make_sc_gather(B, V, D):
    assert D == _NW, (B, V, D)
    mesh = plsc.VectorSubcoreMesh(core_axis_name="c", subcore_axis_name="s")

    @functools.partial(
        pl.kernel,
        mesh=mesh,
        out_type=jax.ShapeDtypeStruct((2 * D, B), jnp.float32),
        scratch_types=[
            pltpu.VMEM((V,), jnp.float32),
            pltpu.VMEM((_CH,), jnp.int32),
            pltpu.VMEM((_CH,), jnp.int32),
            pltpu.VMEM((_CH,), jnp.float32),
            pltpu.VMEM((_CH,), jnp.float32),
            pltpu.SemaphoreType.DMA,
            pltpu.SemaphoreType.DMA,
            pltpu.SemaphoreType.DMA,
        ],
        compiler_params=pltpu.CompilerParams(needs_layout_passes=False),
    )
    def gather_kernel(uidx_hbm, midx_hbm, utabT_hbm, mtabT_hbm, out_hbm,
                      rowbuf, idxbuf0, idxbuf1, outbuf0, outbuf1,
                      row_sem, idx_sem, out_sem):
        idxbufs = (idxbuf0, idxbuf1)
        outbufs = (outbuf0, outbuf1)
        wid = lax.axis_index("s") * _NC + lax.axis_index("c")
        n_ch = B // _CH
        towers = ((utabT_hbm, uidx_hbm), (mtabT_hbm, midx_hbm))
        out_dmas = []
        row_dma = pltpu.async_copy(utabT_hbm.at[wid], rowbuf, row_sem)
        idx_dma = pltpu.async_copy(
            uidx_hbm.at[pl.ds(0, _CH)], idxbuf0, idx_sem)
        for tower, (tab, idx_hbm) in enumerate(towers):
            row_dma.wait()
            for c in range(n_ch):
                idx_dma.wait()
                nxt = c + 1
                if nxt < n_ch:
                    idx_dma = pltpu.async_copy(
                        idx_hbm.at[pl.ds(nxt * _CH, _CH)],
                        idxbufs[nxt % 2], idx_sem)
                elif tower == 0:
                    idx_dma = pltpu.async_copy(
                        midx_hbm.at[pl.ds(0, _CH)], idxbufs[0], idx_sem)
                if len(out_dmas) >= 2:
                    out_dmas.pop(0).wait()
                ibuf = idxbufs[c % 2]
                obuf = outbufs[c % 2]

                def step(i, carry, ibuf=ibuf, obuf=obuf):
                    base = i * 64
                    for j in range(4):
                        iv = ibuf[pl.ds(base + j * 16, 16)]
                        obuf[pl.ds(base + j * 16, 16)] = plsc.load_gather(
                            rowbuf, [iv])
                    return carry

                lax.fori_loop(0, _CH // 64, step, 0)
                if tower == 0 and c == n_ch - 1:
                    row_dma = pltpu.async_copy(
                        mtabT_hbm.at[wid], rowbuf, row_sem)
                out_dmas.append(pltpu.async_copy(
                    obuf, out_hbm.at[tower * D + wid, pl.ds(c * _CH, _CH)],
                    out_sem))
        for dma in out_dmas:
            dma.wait()

    return gather_kernel


def _mlp_t_body(x_ref, w1t_ref, b1_ref, w2t_ref, b2_ref, w3t_ref, b3_ref,
                o_ref):
    h = jnp.dot(w1t_ref[...], x_ref[...], preferred_element_type=jnp.float32)
    h = jnp.maximum(h + b1_ref[...], 0.0)
    h = jnp.dot(w2t_ref[...], h, preferred_element_type=jnp.float32)
    h = jnp.maximum(h + b2_ref[...], 0.0)
    o_ref[...] = (
        jnp.dot(w3t_ref[...], h, preferred_element_type=jnp.float32)
        + b3_ref[...]
    )


@functools.lru_cache(maxsize=None)
def _make_mlp(B, D2, H, BLK):
    full = lambda shape: pl.BlockSpec(shape, lambda i: (0,) * len(shape))
    return pl.pallas_call(
        _mlp_t_body,
        grid=(B // BLK,),
        in_specs=[
            pl.BlockSpec((D2, BLK), lambda i: (0, i)),
            full((H, D2)),
            full((H, 1)),
            full((H, H)),
            full((H, 1)),
            full((1, H)),
            full((1, 1)),
        ],
        out_specs=pl.BlockSpec((1, BLK), lambda i: (0, i)),
        out_shape=jax.ShapeDtypeStruct((1, B), jnp.float32),
    )


def kernel(user, movie, user_table, movie_table, W1, b1, W2, b2, W3, b3):
    B = user.shape[0]
    V, D = user_table.shape
    H = W2.shape[0]
    embT = _make_sc_gather(B, V, D)(user, movie, user_table.T, movie_table.T)
    outT = _make_mlp(B, 2 * D, H, 4096)(
        embT, W1.T, b1.reshape(H, 1), W2.T, b2.reshape(H, 1), W3.T,
        b3.reshape(1, 1))
    return outT.reshape(B, 1)


# speedup vs baseline: 1.0144x; 1.0144x over previous
import functools

import jax
import jax.numpy as jnp
from jax import lax
from jax.experimental import pallas as pl
from jax.experimental.pallas import tpu as pltpu
from jax.experimental.pallas import tpu_sc as plsc

_NC = 2
_NS = 16
_NW = _NC * _NS
_CH = 4096


@functools.lru_cache(maxsize=None)
def _make_sc_gather(B, V, D):
    assert D == _NW, (B, V, D)
    mesh = plsc.VectorSubcoreMesh(core_axis_name="c", subcore_axis_name="s")

    @functools.partial(
        pl.kernel,
        mesh=mesh,
        out_type=jax.ShapeDtypeStruct((2 * D, B), jnp.float32),
        scratch_types=[
            pltpu.VMEM((V,), jnp.float32),
            pltpu.VMEM((_CH,), jnp.int32),
            pltpu.VMEM((_CH,), jnp.int32),
            pltpu.VMEM((_CH,), jnp.float32),
            pltpu.VMEM((_CH,), jnp.float32),
            pltpu.SemaphoreType.DMA,
            pltpu.SemaphoreType.DMA,
            pltpu.SemaphoreType.DMA,
        ],
        compiler_params=pltpu.CompilerParams(needs_layout_passes=False),
    )
    def gather_kernel(uidx_hbm, midx_hbm, utabT_hbm, mtabT_hbm, out_hbm,
                      rowbuf, idxbuf0, idxbuf1, outbuf0, outbuf1,
                      row_sem, idx_sem, out_sem):
        idxbufs = (idxbuf0, idxbuf1)
        outbufs = (outbuf0, outbuf1)
        wid = lax.axis_index("s") * _NC + lax.axis_index("c")
        n_ch = B // _CH
        towers = ((utabT_hbm, uidx_hbm), (mtabT_hbm, midx_hbm))
        out_dmas = []
        row_dma = pltpu.async_copy(utabT_hbm.at[wid], rowbuf, row_sem)
        idx_dma = pltpu.async_copy(
            uidx_hbm.at[pl.ds(0, _CH)], idxbuf0, idx_sem)
        for tower, (tab, idx_hbm) in enumerate(towers):
            row_dma.wait()
            for c in range(n_ch):
                idx_dma.wait()
                nxt = c + 1
                if nxt < n_ch:
                    idx_dma = pltpu.async_copy(
                        idx_hbm.at[pl.ds(nxt * _CH, _CH)],
                        idxbufs[nxt % 2], idx_sem)
                elif tower == 0:
                    idx_dma = pltpu.async_copy(
                        midx_hbm.at[pl.ds(0, _CH)], idxbufs[0], idx_sem)
                if len(out_dmas) >= 2:
                    out_dmas.pop(0).wait()
                ibuf = idxbufs[c % 2]
                obuf = outbufs[c % 2]

                def step(i, carry, ibuf=ibuf, obuf=obuf):
                    base = i * 128
                    for j in range(8):
                        iv = ibuf[pl.ds(base + j * 16, 16)]
                        obuf[pl.ds(base + j * 16, 16)] = plsc.load_gather(
                            rowbuf, [iv])
                    return carry

                lax.fori_loop(0, _CH // 128, step, 0)
                if tower == 0 and c == n_ch - 1:
                    row_dma = pltpu.async_copy(
                        mtabT_hbm.at[wid], rowbuf, row_sem)
                out_dmas.append(pltpu.async_copy(
                    obuf, out_hbm.at[tower * D + wid, pl.ds(c * _CH, _CH)],
                    out_sem))
        for dma in out_dmas:
            dma.wait()

    return gather_kernel


def _mlp_t_body(x_ref, w1t_ref, b1_ref, w2t_ref, b2_ref, w3t_ref, b3_ref,
                o_ref):
    h = jnp.dot(w1t_ref[...], x_ref[...], preferred_element_type=jnp.float32)
    h = jnp.maximum(h + b1_ref[...], 0.0)
    h = jnp.dot(w2t_ref[...], h, preferred_element_type=jnp.float32)
    h = jnp.maximum(h + b2_ref[...], 0.0)
    o_ref[...] = (
        jnp.dot(w3t_ref[...], h, preferred_element_type=jnp.float32)
        + b3_ref[...]
    )


@functools.lru_cache(maxsize=None)
def _make_mlp(B, D2, H, BLK):
    full = lambda shape: pl.BlockSpec(shape, lambda i: (0,) * len(shape))
    return pl.pallas_call(
        _mlp_t_body,
        grid=(B // BLK,),
        in_specs=[
            pl.BlockSpec((D2, BLK), lambda i: (0, i)),
            full((H, D2)),
            full((H, 1)),
            full((H, H)),
            full((H, 1)),
            full((1, H)),
            full((1, 1)),
        ],
        out_specs=pl.BlockSpec((1, BLK), lambda i: (0, i)),
        out_shape=jax.ShapeDtypeStruct((1, B), jnp.float32),
    )


def kernel(user, movie, user_table, movie_table, W1, b1, W2, b2, W3, b3):
    B = user.shape[0]
    V, D = user_table.shape
    H = W2.shape[0]
    embT = _make_sc_gather(B, V, D)(user, movie, user_table.T, movie_table.T)
    outT = _make_mlp(B, 2 * D, H, 8192)(
        embT, W1.T, b1.reshape(H, 1), W2.T, b2.reshape(H, 1), W3.T,
        b3.reshape(1, 1))
    return outT.reshape(B, 1)


# speedup vs baseline: 1.0250x; 1.0104x over previous
import functools

import jax
import jax.numpy as jnp
from jax import lax
from jax.experimental import pallas as pl
from jax.experimental.pallas import tpu as pltpu
from jax.experimental.pallas import tpu_sc as plsc

_NC = 2
_NS = 16
_NW = _NC * _NS
_CH = 4096


@functools.lru_cache(maxsize=None)
def _make_sc_gather(B, V, D):
    assert D == _NW, (B, V, D)
    mesh = plsc.VectorSubcoreMesh(core_axis_name="c", subcore_axis_name="s")

    @functools.partial(
        pl.kernel,
        mesh=mesh,
        out_type=jax.ShapeDtypeStruct((2 * D, B), jnp.float32),
        scratch_types=[
            pltpu.VMEM((V,), jnp.float32),
            pltpu.VMEM((_CH,), jnp.int32),
            pltpu.VMEM((_CH,), jnp.int32),
            pltpu.VMEM((_CH,), jnp.float32),
            pltpu.VMEM((_CH,), jnp.float32),
            pltpu.SemaphoreType.DMA,
            pltpu.SemaphoreType.DMA,
            pltpu.SemaphoreType.DMA,
        ],
        compiler_params=pltpu.CompilerParams(
            needs_layout_passes=False, skip_device_barrier=True),
    )
    def gather_kernel(uidx_hbm, midx_hbm, utabT_hbm, mtabT_hbm, out_hbm,
                      rowbuf, idxbuf0, idxbuf1, outbuf0, outbuf1,
                      row_sem, idx_sem, out_sem):
        idxbufs = (idxbuf0, idxbuf1)
        outbufs = (outbuf0, outbuf1)
        wid = lax.axis_index("s") * _NC + lax.axis_index("c")
        n_ch = B // _CH
        towers = ((utabT_hbm, uidx_hbm), (mtabT_hbm, midx_hbm))
        out_dmas = []
        row_dma = pltpu.async_copy(utabT_hbm.at[wid], rowbuf, row_sem)
        idx_dma = pltpu.async_copy(
            uidx_hbm.at[pl.ds(0, _CH)], idxbuf0, idx_sem)
        for tower, (tab, idx_hbm) in enumerate(towers):
            row_dma.wait()
            for c in range(n_ch):
                idx_dma.wait()
                nxt = c + 1
                if nxt < n_ch:
                    idx_dma = pltpu.async_copy(
                        idx_hbm.at[pl.ds(nxt * _CH, _CH)],
                        idxbufs[nxt % 2], idx_sem)
                elif tower == 0:
                    idx_dma = pltpu.async_copy(
                        midx_hbm.at[pl.ds(0, _CH)], idxbufs[0], idx_sem)
                if len(out_dmas) >= 2:
                    out_dmas.pop(0).wait()
                ibuf = idxbufs[c % 2]
                obuf = outbufs[c % 2]

                def step(i, carry, ibuf=ibuf, obuf=obuf):
                    base = i * 64
                    for j in range(4):
                        iv = ibuf[pl.ds(base + j * 16, 16)]
                        obuf[pl.ds(base + j * 16, 16)] = plsc.load_gather(
                            rowbuf, [iv])
                    return carry

                lax.fori_loop(0, _CH // 64, step, 0)
                if tower == 0 and c == n_ch - 1:
                    row_dma = pltpu.async_copy(
                        mtabT_hbm.at[wid], rowbuf, row_sem)
                out_dmas.append(pltpu.async_copy(
                    obuf, out_hbm.at[tower * D + wid, pl.ds(c * _CH, _CH)],
                    out_sem))
        for dma in out_dmas:
            dma.wait()

    return gather_kernel


def _mlp_t_body(x_ref, w1t_ref, b1_ref, w2t_ref, b2_ref, w3t_ref, b3_ref,
                o_ref):
    h = jnp.dot(w1t_ref[...], x_ref[...], preferred_element_type=jnp.float32)
    h = jnp.maximum(h + b1_ref[...], 0.0)
    h = jnp.dot(w2t_ref[...], h, preferred_element_type=jnp.float32)
    h = jnp.maximum(h + b2_ref[...], 0.0)
    o_ref[...] = (
        jnp.dot(w3t_ref[...], h, preferred_element_type=jnp.float32)
        + b3_ref[...]
    )


@functools.lru_cache(maxsize=None)
def _make_mlp(B, D2, H, BLK):
    full = lambda shape: pl.BlockSpec(shape, lambda i: (0,) * len(shape))
    return pl.pallas_call(
        _mlp_t_body,
        grid=(B // BLK,),
        in_specs=[
            pl.BlockSpec((D2, BLK), lambda i: (0, i)),
            full((H, D2)),
            full((H, 1)),
            full((H, H)),
            full((H, 1)),
            full((1, H)),
            full((1, 1)),
        ],
        out_specs=pl.BlockSpec((1, BLK), lambda i: (0, i)),
        out_shape=jax.ShapeDtypeStruct((1, B), jnp.float32),
        compiler_params=pltpu.CompilerParams(skip_device_barrier=True),
    )


def kernel(user, movie, user_table, movie_table, W1, b1, W2, b2, W3, b3):
    B = user.shape[0]
    V, D = user_table.shape
    H = W2.shape[0]
    embT = _make_sc_gather(B, V, D)(user, movie, user_table.T, movie_table.T)
    outT = _make_mlp(B, 2 * D, H, 8192)(
        embT, W1.T, b1.reshape(H, 1), W2.T, b2.reshape(H, 1), W3.T,
        b3.reshape(1, 1))
    return outT.reshape(B, 1)
